# baseline (speedup 1.0000x reference)
import functools

import jax
import jax.numpy as jnp
from jax.experimental import pallas as pl
from jax.experimental.pallas import tpu as pltpu


def _round_up(n, m):
    return ((n + m - 1) // m) * m


def _head_kernel(x_ref, w_ref, b_ref, o_ref, *, inv_nm1):
    tok_sum = jnp.sum(x_ref[...], axis=1, dtype=jnp.float32)
    avg = (tok_sum - x_ref[:, 0, :]) * inv_nm1
    out = jnp.dot(avg.astype(jnp.bfloat16), w_ref[...],
                  preferred_element_type=jnp.float32)
    o_ref[...] = out + b_ref[...]


def kernel(x, w, b):
    B, S, D = x.shape
    D_in, C = w.shape

    w = w.astype(jnp.bfloat16)
    b2 = b.reshape(1, C)

    TILE_B = 16
    while TILE_B > 1 and B % TILE_B != 0:
        TILE_B //= 2
    nb = B // TILE_B

    itemsize = x.dtype.itemsize
    cost = pl.CostEstimate(
        flops=2 * B * D_in * C + B * S * D,
        transcendentals=0,
        bytes_accessed=(B * S * D * itemsize
                        + D_in * C * w.dtype.itemsize
                        + B * C * 4),
    )
    out = pl.pallas_call(
        functools.partial(_head_kernel, inv_nm1=1.0 / (S - 1)),
        out_shape=jax.ShapeDtypeStruct((B, C), jnp.float32),
        grid=(nb,),
        in_specs=[
            pl.BlockSpec((TILE_B, S, D), lambda i: (i, 0, 0)),
            pl.BlockSpec((D_in, C), lambda i: (0, 0)),
            pl.BlockSpec((1, C), lambda i: (0, 0)),
        ],
        out_specs=pl.BlockSpec((TILE_B, C), lambda i: (i, 0)),
        compiler_params=pltpu.CompilerParams(
            dimension_semantics=("parallel",),
            vmem_limit_bytes=48 * 1024 * 1024,
        ),
        cost_estimate=cost,
    )(x, w, b2)

    return out


# speedup vs baseline: 1.0485x; 1.0485x over previous
import functools

import jax
import jax.numpy as jnp
from jax.experimental import pallas as pl
from jax.experimental.pallas import tpu as pltpu


def _round_up(n, m):
    return ((n + m - 1) // m) * m


def _head_kernel(x_ref, w_ref, b_ref, o_ref, *, inv_nm1):
    tok_sum = jnp.sum(x_ref[...], axis=1, dtype=jnp.float32)
    avg = (tok_sum - x_ref[:, 0, :]) * inv_nm1
    out = jnp.dot(avg, w_ref[...], preferred_element_type=jnp.float32)
    o_ref[...] = out + b_ref[...]


def kernel(x, w, b):
    B, S, D = x.shape
    D_in, C = w.shape

    b2 = b.reshape(1, C)

    TILE_B = 16
    while TILE_B > 1 and B % TILE_B != 0:
        TILE_B //= 2
    nb = B // TILE_B

    itemsize = x.dtype.itemsize
    cost = pl.CostEstimate(
        flops=2 * B * D_in * C + B * S * D,
        transcendentals=0,
        bytes_accessed=(B * S * D * itemsize
                        + D_in * C * w.dtype.itemsize
                        + B * C * 4),
    )
    out = pl.pallas_call(
        functools.partial(_head_kernel, inv_nm1=1.0 / (S - 1)),
        out_shape=jax.ShapeDtypeStruct((B, C), jnp.float32),
        grid=(nb,),
        in_specs=[
            pl.BlockSpec((TILE_B, S, D), lambda i: (i, 0, 0)),
            pl.BlockSpec((D_in, C), lambda i: (0, 0)),
            pl.BlockSpec((1, C), lambda i: (0, 0)),
        ],
        out_specs=pl.BlockSpec((TILE_B, C), lambda i: (i, 0)),
        compiler_params=pltpu.CompilerParams(
            dimension_semantics=("parallel",),
            vmem_limit_bytes=48 * 1024 * 1024,
        ),
        cost_estimate=cost,
    )(x, w, b2)

    return out


# speedup vs baseline: 1.0502x; 1.0016x over previous
import functools

import jax
import jax.numpy as jnp
from jax.experimental import pallas as pl
from jax.experimental.pallas import tpu as pltpu


def _head_kernel(x_ref, w_ref, b_ref, o_ref, *, inv_nm1, seq, dim):
    xt = x_ref[...].reshape(-1, seq, dim)
    tok_sum = jnp.sum(xt, axis=1, dtype=jnp.float32)
    avg = (tok_sum - xt[:, 0, :]) * inv_nm1
    out = jnp.dot(avg, w_ref[...], preferred_element_type=jnp.float32)
    o_ref[...] = out + b_ref[...]


def kernel(x, w, b):
    B, S, D = x.shape
    D_in, C = w.shape
    x2 = x.reshape(B * S, D)
    b2 = b.reshape(1, C)

    TILE_B = 16
    while TILE_B > 1 and B % TILE_B != 0:
        TILE_B //= 2
    nb = B // TILE_B

    itemsize = x.dtype.itemsize
    cost = pl.CostEstimate(
        flops=2 * B * D_in * C + B * S * D,
        transcendentals=0,
        bytes_accessed=(B * S * D * itemsize
                        + D_in * C * w.dtype.itemsize
                        + B * C * 4),
    )
    out = pl.pallas_call(
        functools.partial(_head_kernel, inv_nm1=1.0 / (S - 1), seq=S, dim=D),
        out_shape=jax.ShapeDtypeStruct((B, C), jnp.float32),
        grid=(nb,),
        in_specs=[
            pl.BlockSpec((TILE_B * S, D), lambda i: (i, 0)),
            pl.BlockSpec((D_in, C), lambda i: (0, 0)),
            pl.BlockSpec((1, C), lambda i: (0, 0)),
        ],
        out_specs=pl.BlockSpec((TILE_B, C), lambda i: (i, 0)),
        compiler_params=pltpu.CompilerParams(
            dimension_semantics=("parallel",),
            vmem_limit_bytes=48 * 1024 * 1024,
        ),
        cost_estimate=cost,
    )(x2, w, b2)

    return out
